# baseline (speedup 1.0000x reference)
import functools

import jax
import jax.numpy as jnp
from jax import lax
from jax.experimental import pallas as pl
from jax.experimental.pallas import tpu as pltpu
from jax.experimental.pallas import tpu_sc as plsc

IDENTITIES = 1000
LENGTH = 320
WEIGHT_DIM = 150
BATCH = 1024

NUM_CORES = 2
NUM_SUBCORES = 16
NUM_WORKERS = NUM_CORES * NUM_SUBCORES

LG = LENGTH // 8
UNITS = LG * WEIGHT_DIM
UNITS_PER_WORKER = -(-UNITS // NUM_WORKERS)
NBUF = 3
LOOP_UNITS = -(-UNITS_PER_WORKER // NBUF) * NBUF


def _sc_gather(wt, idx):
    mesh = plsc.VectorSubcoreMesh(core_axis_name="c", subcore_axis_name="s")

    @functools.partial(
        pl.kernel,
        mesh=mesh,
        out_type=jax.ShapeDtypeStruct((WEIGHT_DIM, LENGTH, BATCH), jnp.float32),
        compiler_params=pltpu.CompilerParams(needs_layout_passes=False),
        scratch_types=[
            pltpu.VMEM((BATCH,), jnp.int32),
            pltpu.VMEM((NBUF, 8, IDENTITIES), jnp.float32),
            pltpu.VMEM((NBUF, 8, BATCH), jnp.float32),
        ] + [pltpu.SemaphoreType.DMA] * (2 * NBUF),
    )
    def k(wt_hbm, idx_hbm, out_hbm, idx_v, sbuf, obuf, *sems):
        wid = lax.axis_index("s") * NUM_CORES + lax.axis_index("c")
        pltpu.sync_copy(idx_hbm, idx_v)
        sems_in = sems[:NBUF]
        sems_out = sems[NBUF:]

        def unit_dl(kk):
            u = wid + kk * NUM_WORKERS
            lg = lax.shift_right_logical(u * 55925, 23)
            return u, u - lg * WEIGHT_DIM, lg

        def start_in(kk, p):
            u, d, lg = unit_dl(kk)

            @pl.when(u < UNITS)
            def _():
                pltpu.async_copy(
                    wt_hbm.at[pl.ds(d, 1), pl.ds(lg * 8, 8), :],
                    sbuf.at[pl.ds(p, 1)], sems_in[p])

        def wait_in(kk, p):
            u, d, lg = unit_dl(kk)

            @pl.when(u < UNITS)
            def _():
                pltpu.make_async_copy(
                    wt_hbm.at[pl.ds(d, 1), pl.ds(lg * 8, 8), :],
                    sbuf.at[pl.ds(p, 1)], sems_in[p]).wait()

        def start_out(kk, p):
            u, d, lg = unit_dl(kk)

            @pl.when(u < UNITS)
            def _():
                pltpu.async_copy(
                    obuf.at[pl.ds(p, 1)],
                    out_hbm.at[pl.ds(d, 1), pl.ds(lg * 8, 8), :], sems_out[p])

        def wait_out(kk, p):
            u, d, lg = unit_dl(kk)

            @pl.when(u < UNITS)
            def _():
                pltpu.make_async_copy(
                    obuf.at[pl.ds(p, 1)],
                    out_hbm.at[pl.ds(d, 1), pl.ds(lg * 8, 8), :],
                    sems_out[p]).wait()

        def compute(p):
            pv = jnp.full((16,), p, jnp.int32)
            for bg in range(8):
                for t in range(8):
                    base = bg * 128 + t * 16
                    iv = idx_v[pl.ds(base, 16)]
                    vals = [
                        plsc.load_gather(
                            sbuf, [pv, jnp.full((16,), s, jnp.int32), iv])
                        for s in range(8)
                    ]
                    for s in range(8):
                        obuf[p, s, pl.ds(base, 16)] = vals[s]

        for p in range(NBUF):
            start_in(p, p)

        @pl.loop(0, LOOP_UNITS // NBUF)
        def _group(kkg):
            for p in range(NBUF):
                kk = kkg * NBUF + p
                wait_in(kk, p)

                @pl.when(kkg > 0)
                def _():
                    wait_out(kk - NBUF, p)

                compute(p)
                start_out(kk, p)
                start_in(kk + NBUF, p)

        for p in range(NBUF):
            wait_out(LOOP_UNITS - NBUF + p, p)

    return k(wt, idx)


def kernel(weights, ref_img):
    wt = jnp.transpose(weights, (2, 1, 0))
    idx = ref_img.astype(jnp.int32)
    out_t = _sc_gather(wt, idx)
    return jnp.transpose(out_t, (1, 2, 0))


# speedup vs baseline: 1.0398x; 1.0398x over previous
import functools

import jax
import jax.numpy as jnp
from jax import lax
from jax.experimental import pallas as pl
from jax.experimental.pallas import tpu as pltpu
from jax.experimental.pallas import tpu_sc as plsc

IDENTITIES = 1000
LENGTH = 320
WEIGHT_DIM = 150
BATCH = 1024

NUM_CORES = 2
NUM_SUBCORES = 16
NUM_WORKERS = NUM_CORES * NUM_SUBCORES

LG = LENGTH // 8
UNITS = LG * WEIGHT_DIM
UNITS_PER_WORKER = -(-UNITS // NUM_WORKERS)


def _sc_gather(wt, idx):
    mesh = plsc.VectorSubcoreMesh(core_axis_name="c", subcore_axis_name="s")

    @functools.partial(
        pl.kernel,
        mesh=mesh,
        out_type=jax.ShapeDtypeStruct((WEIGHT_DIM, LENGTH, BATCH), jnp.float32),
        compiler_params=pltpu.CompilerParams(needs_layout_passes=False),
        scratch_types=[
            pltpu.VMEM((BATCH,), jnp.int32),
            pltpu.VMEM((1, 8, IDENTITIES), jnp.float32),
            pltpu.VMEM((1, 8, IDENTITIES), jnp.float32),
            pltpu.VMEM((1, 8, BATCH), jnp.float32),
            pltpu.VMEM((1, 8, BATCH), jnp.float32),
            pltpu.SemaphoreType.DMA,
            pltpu.SemaphoreType.DMA,
            pltpu.SemaphoreType.DMA,
            pltpu.SemaphoreType.DMA,
        ],
    )
    def k(wt_hbm, idx_hbm, out_hbm, idx_v, sbuf0, sbuf1, obuf0, obuf1,
          sem_in0, sem_in1, sem_out0, sem_out1):
        wid = lax.axis_index("s") * NUM_CORES + lax.axis_index("c")
        pltpu.sync_copy(idx_hbm, idx_v)
        sems_in = (sem_in0, sem_in1)
        sems_out = (sem_out0, sem_out1)
        sbufs = (sbuf0, sbuf1)
        obufs = (obuf0, obuf1)

        def unit_dl(kk):
            u = wid + kk * NUM_WORKERS
            lg = lax.shift_right_logical(u * 55925, 23)
            return u, u - lg * WEIGHT_DIM, lg

        def start_in(kk, p):
            u, d, lg = unit_dl(kk)

            @pl.when(u < UNITS)
            def _():
                pltpu.async_copy(
                    wt_hbm.at[pl.ds(d, 1), pl.ds(lg * 8, 8), :],
                    sbufs[p], sems_in[p])

        def wait_in(kk, p):
            u, d, lg = unit_dl(kk)

            @pl.when(u < UNITS)
            def _():
                pltpu.make_async_copy(
                    wt_hbm.at[pl.ds(d, 1), pl.ds(lg * 8, 8), :],
                    sbufs[p], sems_in[p]).wait()

        def start_out(kk, p):
            u, d, lg = unit_dl(kk)

            @pl.when(u < UNITS)
            def _():
                pltpu.async_copy(
                    obufs[p],
                    out_hbm.at[pl.ds(d, 1), pl.ds(lg * 8, 8), :], sems_out[p])

        def wait_out(kk, p):
            u, d, lg = unit_dl(kk)

            @pl.when(u < UNITS)
            def _():
                pltpu.make_async_copy(
                    obufs[p],
                    out_hbm.at[pl.ds(d, 1), pl.ds(lg * 8, 8), :],
                    sems_out[p]).wait()

        def compute(p):
            zv = jnp.zeros((16,), jnp.int32)
            svs = [jnp.full((16,), s, jnp.int32) for s in range(8)]
            for bg in range(8):
                for t in range(8):
                    base = bg * 128 + t * 16
                    iv = idx_v[pl.ds(base, 16)]
                    vals = [
                        plsc.load_gather(sbufs[p], [zv, svs[s], iv])
                        for s in range(8)
                    ]
                    for s in range(8):
                        obufs[p][0, s, pl.ds(base, 16)] = vals[s]

        start_in(0, 0)
        start_in(1, 1)

        @pl.loop(0, UNITS_PER_WORKER // 2)
        def _pair(kk2):
            for p in range(2):
                kk = kk2 * 2 + p
                wait_in(kk, p)

                @pl.when(kk2 > 0)
                def _():
                    wait_out(kk - 2, p)

                compute(p)
                start_out(kk, p)
                start_in(kk + 2, p)

        for p in range(2):
            wait_out(UNITS_PER_WORKER - 2 + p, p)

    return k(wt, idx)


def kernel(weights, ref_img):
    wt = jnp.transpose(weights, (2, 1, 0))
    idx = ref_img.astype(jnp.int32)
    out_t = _sc_gather(wt, idx)
    return jnp.transpose(out_t, (1, 2, 0))
